# baseline (speedup 1.0000x reference)
import functools

import jax
import jax.numpy as jnp
from jax import lax
from jax.experimental import pallas as pl
from jax.experimental.pallas import tpu as pltpu
from jax.experimental.pallas import tpu_sc as plsc

N = 10000
E = 320000
D_IN = 128
D_H = 32

NC = 2
NS = 16
NW = NC * NS

BLK = 2560
BPW = 4
TOTB = NW * BPW
EPAD = TOTB * BLK
NROWS = 10112
RPT = NROWS // NS
TRASH = N
DEGW = 16

_f32 = jnp.float32
_bf16 = jnp.bfloat16
_mesh = plsc.VectorSubcoreMesh(core_axis_name="c", subcore_axis_name="s")
_sc_params = pltpu.CompilerParams(use_tc_tiling_on_sc=False)


def _deg_body(dstb, zdeg, ones_hbm, out, idx_d, ones_v, acc):
    c = lax.axis_index("c")
    s = lax.axis_index("s")
    w = c * NS + s
    r0 = s * RPT
    pltpu.sync_copy(zdeg.at[pl.ds(r0, RPT)], acc.at[pl.ds(r0, RPT)])
    pltpu.sync_copy(ones_hbm, ones_v)
    pltpu.sync_copy(dstb.at[pl.ds(w * BPW, BPW)], idx_d)
    plsc.subcore_barrier()

    def body(j, carry):
        pltpu.sync_copy(ones_v, acc.at[idx_d.at[j]], add=True)
        return carry

    lax.fori_loop(0, BPW, body, 0, unroll=False)
    plsc.subcore_barrier()
    pltpu.sync_copy(acc.at[pl.ds(r0, RPT)], out.at[c, pl.ds(r0, RPT)])


_deg_call = functools.partial(
    pl.kernel,
    out_type=jax.ShapeDtypeStruct((NC, NROWS, DEGW), _bf16),
    mesh=_mesh,
    compiler_params=_sc_params,
    scratch_types=[
        pltpu.VMEM((BPW, BLK), jnp.int32),
        pltpu.VMEM((BLK, DEGW), _bf16),
        pltpu.VMEM_SHARED((NROWS, DEGW), _bf16),
    ],
)(_deg_body)


BPW0 = 3
BPW1 = BPW * 2 - BPW0
BPWMAX = max(BPW0, BPW1)


def _agg_body(g_hbm, srcb, dstb, zacc, out, idx_s, idx_d, rows, acc, sem):
    c = lax.axis_index("c")
    s = lax.axis_index("s")
    r0 = s * RPT
    b0 = jnp.where(c == 0, s * BPW0, NS * BPW0 + s * BPW1)
    nb = jnp.where(c == 0, BPW0, BPW1)
    pltpu.sync_copy(zacc.at[pl.ds(r0, RPT)], acc.at[pl.ds(r0, RPT)])
    pltpu.sync_copy(srcb.at[pl.ds(b0, BPWMAX)], idx_s)
    pltpu.sync_copy(dstb.at[pl.ds(b0, BPWMAX)], idx_d)
    plsc.subcore_barrier()

    def body(j, carry):
        pltpu.async_copy(g_hbm.at[idx_s.at[j]], rows, sem).wait()
        pltpu.sync_copy(rows, acc.at[idx_d.at[j]], add=True)
        return carry

    lax.fori_loop(0, nb, body, 0, unroll=False)
    plsc.subcore_barrier()
    pltpu.sync_copy(acc.at[pl.ds(r0, RPT)], out.at[c, pl.ds(r0, RPT)])


_agg_call = functools.partial(
    pl.kernel,
    out_type=jax.ShapeDtypeStruct((NC, NROWS, D_H), _bf16),
    mesh=_mesh,
    compiler_params=_sc_params,
    scratch_types=[
        pltpu.VMEM((BPWMAX, BLK), jnp.int32),
        pltpu.VMEM((BPWMAX, BLK), jnp.int32),
        pltpu.VMEM((BLK, D_H), _bf16),
        pltpu.VMEM_SHARED((NROWS, D_H), _bf16),
        pltpu.SemaphoreType.DMA,
    ],
)(_agg_body)


def _g_body(x_ref, w_ref, degp_ref, g_ref, dinv_ref):
    dsum = degp_ref[0].astype(_f32) + degp_ref[1].astype(_f32)
    deg = jnp.sum(dsum[:N], axis=1, keepdims=True) * (1.0 / DEGW) + 1.0
    dinv = lax.rsqrt(jnp.maximum(deg, 1.0))
    dinv_ref[...] = dinv
    h1 = jnp.dot(x_ref[...], w_ref[...], preferred_element_type=_f32)
    g_ref[...] = (h1 * dinv).astype(_bf16)


def _g_call(x, W1, degp):
    return pl.pallas_call(
        _g_body,
        out_shape=(
            jax.ShapeDtypeStruct((N, D_H), _bf16),
            jax.ShapeDtypeStruct((N, 1), _f32),
        ),
    )(x, W1, degp)


def _mid_body(p_ref, g_ref, dinv_ref, b_ref, w_ref, o_ref):
    p = p_ref[0].astype(_f32)[:N] + p_ref[1].astype(_f32)[:N]
    agg = p + g_ref[...].astype(_f32)
    out1 = agg * dinv_ref[...] + b_ref[...]
    h2 = jnp.maximum(out1, 0.0)
    g2 = jnp.dot(h2, w_ref[...], preferred_element_type=_f32) * dinv_ref[...]
    o_ref[...] = g2.astype(_bf16)


def _mid_call(p1, g1, dinv, b1, W2):
    return pl.pallas_call(
        _mid_body,
        out_shape=jax.ShapeDtypeStruct((N, D_H), _bf16),
    )(p1, g1, dinv, b1.reshape(1, D_H), W2)


def _out_body(p_ref, g_ref, dinv_ref, b_ref, o_ref):
    p = p_ref[0].astype(_f32)[:N] + p_ref[1].astype(_f32)[:N]
    agg = p + g_ref[...].astype(_f32)
    o_ref[...] = agg * dinv_ref[...] + b_ref[...]


def _out_call(p2, g2, dinv, b2):
    return pl.pallas_call(
        _out_body,
        out_shape=jax.ShapeDtypeStruct((N, D_H), _f32),
    )(p2, g2, dinv, b2.reshape(1, D_H))


def kernel(x, edge_index, W1, b1, W2, b2):
    pad = EPAD - E
    srcp = jnp.concatenate(
        [edge_index[0], jnp.zeros((pad,), jnp.int32)]).reshape(TOTB, BLK)
    trash = TRASH + jnp.arange(pad, dtype=jnp.int32) % (NROWS - N)
    dstp = jnp.concatenate([edge_index[1], trash]).reshape(TOTB, BLK)
    zacc = jnp.zeros((NROWS, D_H), _bf16)
    zdeg = jnp.zeros((NROWS, DEGW), _bf16)
    ones = jnp.ones((BLK, DEGW), _bf16)

    degp = _deg_call(dstp, zdeg, ones)
    g1, dinv = _g_call(x, W1, degp)
    p1 = _agg_call(g1, srcp, dstp, zacc)
    g2 = _mid_call(p1, g1, dinv, b1, W2)
    p2 = _agg_call(g2, srcp, dstp, zacc)
    return _out_call(p2, g2, dinv, b2)


# speedup vs baseline: 1.0736x; 1.0736x over previous
import functools

import jax
import jax.numpy as jnp
from jax import lax
from jax.experimental import pallas as pl
from jax.experimental.pallas import tpu as pltpu
from jax.experimental.pallas import tpu_sc as plsc

N = 10000
E = 320000
D_IN = 128
D_H = 32

NC = 2
NS = 16
NW = NC * NS

BLK = 2560
BPW = 4
TOTB = NW * BPW
EPAD = TOTB * BLK
NROWS = 10112
RPT = NROWS // NS
TRASH = N
DEGW = 16

_f32 = jnp.float32
_bf16 = jnp.bfloat16
_mesh = plsc.VectorSubcoreMesh(core_axis_name="c", subcore_axis_name="s")
_sc_params = pltpu.CompilerParams(use_tc_tiling_on_sc=False)


def _deg_body(dstb, zdeg, ones_hbm, out, idx_d, ones_v, acc):
    c = lax.axis_index("c")
    s = lax.axis_index("s")
    w = c * NS + s
    r0 = s * RPT
    pltpu.sync_copy(zdeg.at[pl.ds(r0, RPT)], acc.at[pl.ds(r0, RPT)])
    pltpu.sync_copy(ones_hbm, ones_v)
    pltpu.sync_copy(dstb.at[pl.ds(w * BPW, BPW)], idx_d)
    plsc.subcore_barrier()

    def body(j, carry):
        pltpu.sync_copy(ones_v, acc.at[idx_d.at[j]], add=True)
        return carry

    lax.fori_loop(0, BPW, body, 0, unroll=False)
    plsc.subcore_barrier()
    pltpu.sync_copy(acc.at[pl.ds(r0, RPT)], out.at[c, pl.ds(r0, RPT)])


_deg_call = functools.partial(
    pl.kernel,
    out_type=jax.ShapeDtypeStruct((NC, NROWS, DEGW), _bf16),
    mesh=_mesh,
    compiler_params=_sc_params,
    scratch_types=[
        pltpu.VMEM((BPW, BLK), jnp.int32),
        pltpu.VMEM((BLK, DEGW), _bf16),
        pltpu.VMEM_SHARED((NROWS, DEGW), _bf16),
    ],
)(_deg_body)


BPW0 = 5
BPW1 = BPW * 2 - BPW0
BPWMAX = max(BPW0, BPW1)


def _agg_body(g_hbm, srcb, dstb, zacc, out, idx_s, idx_d, rows, acc, sem):
    c = lax.axis_index("c")
    s = lax.axis_index("s")
    r0 = s * RPT
    b0 = jnp.where(c == 0, s * BPW0, NS * BPW0 + s * BPW1)
    nb = jnp.where(c == 0, BPW0, BPW1)
    pltpu.sync_copy(zacc.at[pl.ds(r0, RPT)], acc.at[pl.ds(r0, RPT)])
    pltpu.sync_copy(srcb.at[pl.ds(b0, BPWMAX)], idx_s)
    pltpu.sync_copy(dstb.at[pl.ds(b0, BPWMAX)], idx_d)
    plsc.subcore_barrier()

    def body(j, carry):
        pltpu.async_copy(g_hbm.at[idx_s.at[j]], rows, sem).wait()
        pltpu.sync_copy(rows, acc.at[idx_d.at[j]], add=True)
        return carry

    lax.fori_loop(0, nb, body, 0, unroll=False)
    plsc.subcore_barrier()
    pltpu.sync_copy(acc.at[pl.ds(r0, RPT)], out.at[c, pl.ds(r0, RPT)])


_agg_call = functools.partial(
    pl.kernel,
    out_type=jax.ShapeDtypeStruct((NC, NROWS, D_H), _bf16),
    mesh=_mesh,
    compiler_params=_sc_params,
    scratch_types=[
        pltpu.VMEM((BPWMAX, BLK), jnp.int32),
        pltpu.VMEM((BPWMAX, BLK), jnp.int32),
        pltpu.VMEM((BLK, D_H), _bf16),
        pltpu.VMEM_SHARED((NROWS, D_H), _bf16),
        pltpu.SemaphoreType.DMA,
    ],
)(_agg_body)


def _g_body(x_ref, w_ref, degp_ref, g_ref, dinv_ref):
    dsum = degp_ref[0].astype(_f32) + degp_ref[1].astype(_f32)
    deg = jnp.sum(dsum[:N], axis=1, keepdims=True) * (1.0 / DEGW) + 1.0
    dinv = lax.rsqrt(jnp.maximum(deg, 1.0))
    dinv_ref[...] = dinv
    h1 = jnp.dot(x_ref[...], w_ref[...], preferred_element_type=_f32)
    g_ref[...] = (h1 * dinv).astype(_bf16)


def _g_call(x, W1, degp):
    return pl.pallas_call(
        _g_body,
        out_shape=(
            jax.ShapeDtypeStruct((N, D_H), _bf16),
            jax.ShapeDtypeStruct((N, 1), _f32),
        ),
    )(x, W1, degp)


def _mid_body(p_ref, g_ref, dinv_ref, b_ref, w_ref, o_ref):
    p = p_ref[0].astype(_f32)[:N] + p_ref[1].astype(_f32)[:N]
    agg = p + g_ref[...].astype(_f32)
    out1 = agg * dinv_ref[...] + b_ref[...]
    h2 = jnp.maximum(out1, 0.0)
    g2 = jnp.dot(h2, w_ref[...], preferred_element_type=_f32) * dinv_ref[...]
    o_ref[...] = g2.astype(_bf16)


def _mid_call(p1, g1, dinv, b1, W2):
    return pl.pallas_call(
        _mid_body,
        out_shape=jax.ShapeDtypeStruct((N, D_H), _bf16),
    )(p1, g1, dinv, b1.reshape(1, D_H), W2)


def _out_body(p_ref, g_ref, dinv_ref, b_ref, o_ref):
    p = p_ref[0].astype(_f32)[:N] + p_ref[1].astype(_f32)[:N]
    agg = p + g_ref[...].astype(_f32)
    o_ref[...] = agg * dinv_ref[...] + b_ref[...]


def _out_call(p2, g2, dinv, b2):
    return pl.pallas_call(
        _out_body,
        out_shape=jax.ShapeDtypeStruct((N, D_H), _f32),
    )(p2, g2, dinv, b2.reshape(1, D_H))


def kernel(x, edge_index, W1, b1, W2, b2):
    pad = EPAD - E
    srcp = jnp.concatenate(
        [edge_index[0], jnp.zeros((pad,), jnp.int32)]).reshape(TOTB, BLK)
    trash = TRASH + jnp.arange(pad, dtype=jnp.int32) % (NROWS - N)
    dstp = jnp.concatenate([edge_index[1], trash]).reshape(TOTB, BLK)
    zacc = jnp.zeros((NROWS, D_H), _bf16)
    zdeg = jnp.zeros((NROWS, DEGW), _bf16)
    ones = jnp.ones((BLK, DEGW), _bf16)

    degp = _deg_call(dstp, zdeg, ones)
    g1, dinv = _g_call(x, W1, degp)
    p1 = _agg_call(g1, srcp, dstp, zacc)
    g2 = _mid_call(p1, g1, dinv, b1, W2)
    p2 = _agg_call(g2, srcp, dstp, zacc)
    return _out_call(p2, g2, dinv, b2)


# speedup vs baseline: 1.1170x; 1.0404x over previous
import functools

import jax
import jax.numpy as jnp
from jax import lax
from jax.experimental import pallas as pl
from jax.experimental.pallas import tpu as pltpu
from jax.experimental.pallas import tpu_sc as plsc

N = 10000
E = 320000
D_IN = 128
D_H = 32

NC = 2
NS = 16
NW = NC * NS

BLK = 2560
BPW = 4
TOTB = NW * BPW
EPAD = TOTB * BLK
NROWS = 10112
RPT = NROWS // NS
TRASH = N
DEGW = 16

_f32 = jnp.float32
_bf16 = jnp.bfloat16
_mesh = plsc.VectorSubcoreMesh(core_axis_name="c", subcore_axis_name="s")
_sc_params = pltpu.CompilerParams(use_tc_tiling_on_sc=False)


def _deg_body(dstb, zdeg, ones_hbm, out, idx_d, ones_v, acc):
    c = lax.axis_index("c")
    s = lax.axis_index("s")
    w = c * NS + s
    r0 = s * RPT
    pltpu.sync_copy(zdeg.at[pl.ds(r0, RPT)], acc.at[pl.ds(r0, RPT)])
    pltpu.sync_copy(ones_hbm, ones_v)
    pltpu.sync_copy(dstb.at[pl.ds(w * BPW, BPW)], idx_d)
    plsc.subcore_barrier()

    def body(j, carry):
        pltpu.sync_copy(ones_v, acc.at[idx_d.at[j]], add=True)
        return carry

    lax.fori_loop(0, BPW, body, 0, unroll=False)
    plsc.subcore_barrier()
    pltpu.sync_copy(acc.at[pl.ds(r0, RPT)], out.at[c, pl.ds(r0, RPT)])


_deg_call = functools.partial(
    pl.kernel,
    out_type=jax.ShapeDtypeStruct((NC, NROWS, DEGW), _bf16),
    mesh=_mesh,
    compiler_params=_sc_params,
    scratch_types=[
        pltpu.VMEM((BPW, BLK), jnp.int32),
        pltpu.VMEM((BLK, DEGW), _bf16),
        pltpu.VMEM_SHARED((NROWS, DEGW), _bf16),
    ],
)(_deg_body)


BPW0 = 6
BPW1 = BPW * 2 - BPW0
BPWMAX = max(BPW0, BPW1)


def _agg_body(g_hbm, srcb, dstb, zacc, out, idx_s, idx_d, rows, acc, sem):
    c = lax.axis_index("c")
    s = lax.axis_index("s")
    r0 = s * RPT
    b0 = jnp.where(c == 0, s * BPW0, NS * BPW0 + s * BPW1)
    nb = jnp.where(c == 0, BPW0, BPW1)
    pltpu.sync_copy(zacc.at[pl.ds(r0, RPT)], acc.at[pl.ds(r0, RPT)])
    pltpu.sync_copy(srcb.at[pl.ds(b0, BPWMAX)], idx_s)
    pltpu.sync_copy(dstb.at[pl.ds(b0, BPWMAX)], idx_d)
    plsc.subcore_barrier()

    def body(j, carry):
        pltpu.async_copy(g_hbm.at[idx_s.at[j]], rows, sem).wait()
        pltpu.sync_copy(rows, acc.at[idx_d.at[j]], add=True)
        return carry

    lax.fori_loop(0, nb, body, 0, unroll=False)
    plsc.subcore_barrier()
    pltpu.sync_copy(acc.at[pl.ds(r0, RPT)], out.at[c, pl.ds(r0, RPT)])


_agg_call = functools.partial(
    pl.kernel,
    out_type=jax.ShapeDtypeStruct((NC, NROWS, D_H), _bf16),
    mesh=_mesh,
    compiler_params=_sc_params,
    scratch_types=[
        pltpu.VMEM((BPWMAX, BLK), jnp.int32),
        pltpu.VMEM((BPWMAX, BLK), jnp.int32),
        pltpu.VMEM((BLK, D_H), _bf16),
        pltpu.VMEM_SHARED((NROWS, D_H), _bf16),
        pltpu.SemaphoreType.DMA,
    ],
)(_agg_body)


def _g_body(x_ref, w_ref, degp_ref, g_ref, dinv_ref):
    dsum = degp_ref[0].astype(_f32) + degp_ref[1].astype(_f32)
    deg = jnp.sum(dsum[:N], axis=1, keepdims=True) * (1.0 / DEGW) + 1.0
    dinv = lax.rsqrt(jnp.maximum(deg, 1.0))
    dinv_ref[...] = dinv
    h1 = jnp.dot(x_ref[...], w_ref[...], preferred_element_type=_f32)
    g_ref[...] = (h1 * dinv).astype(_bf16)


def _g_call(x, W1, degp):
    return pl.pallas_call(
        _g_body,
        out_shape=(
            jax.ShapeDtypeStruct((N, D_H), _bf16),
            jax.ShapeDtypeStruct((N, 1), _f32),
        ),
    )(x, W1, degp)


def _mid_body(p_ref, g_ref, dinv_ref, b_ref, w_ref, o_ref):
    p = p_ref[0].astype(_f32)[:N] + p_ref[1].astype(_f32)[:N]
    agg = p + g_ref[...].astype(_f32)
    out1 = agg * dinv_ref[...] + b_ref[...]
    h2 = jnp.maximum(out1, 0.0)
    g2 = jnp.dot(h2, w_ref[...], preferred_element_type=_f32) * dinv_ref[...]
    o_ref[...] = g2.astype(_bf16)


def _mid_call(p1, g1, dinv, b1, W2):
    return pl.pallas_call(
        _mid_body,
        out_shape=jax.ShapeDtypeStruct((N, D_H), _bf16),
    )(p1, g1, dinv, b1.reshape(1, D_H), W2)


def _out_body(p_ref, g_ref, dinv_ref, b_ref, o_ref):
    p = p_ref[0].astype(_f32)[:N] + p_ref[1].astype(_f32)[:N]
    agg = p + g_ref[...].astype(_f32)
    o_ref[...] = agg * dinv_ref[...] + b_ref[...]


def _out_call(p2, g2, dinv, b2):
    return pl.pallas_call(
        _out_body,
        out_shape=jax.ShapeDtypeStruct((N, D_H), _f32),
    )(p2, g2, dinv, b2.reshape(1, D_H))


def kernel(x, edge_index, W1, b1, W2, b2):
    pad = EPAD - E
    srcp = jnp.concatenate(
        [edge_index[0], jnp.zeros((pad,), jnp.int32)]).reshape(TOTB, BLK)
    trash = TRASH + jnp.arange(pad, dtype=jnp.int32) % (NROWS - N)
    dstp = jnp.concatenate([edge_index[1], trash]).reshape(TOTB, BLK)
    zacc = jnp.zeros((NROWS, D_H), _bf16)
    zdeg = jnp.zeros((NROWS, DEGW), _bf16)
    ones = jnp.ones((BLK, DEGW), _bf16)

    degp = _deg_call(dstp, zdeg, ones)
    g1, dinv = _g_call(x, W1, degp)
    p1 = _agg_call(g1, srcp, dstp, zacc)
    g2 = _mid_call(p1, g1, dinv, b1, W2)
    p2 = _agg_call(g2, srcp, dstp, zacc)
    return _out_call(p2, g2, dinv, b2)


# speedup vs baseline: 1.1184x; 1.0013x over previous
import functools

import jax
import jax.numpy as jnp
from jax import lax
from jax.experimental import pallas as pl
from jax.experimental.pallas import tpu as pltpu
from jax.experimental.pallas import tpu_sc as plsc

N = 10000
E = 320000
D_IN = 128
D_H = 32

NC = 2
NS = 16
NW = NC * NS

BLK = 2560
BPW = 4
TOTB = NW * BPW
EPAD = TOTB * BLK
NROWS = 10112
RPT = NROWS // NS
TRASH = N
DEGW = 16

_f32 = jnp.float32
_bf16 = jnp.bfloat16
_mesh = plsc.VectorSubcoreMesh(core_axis_name="c", subcore_axis_name="s")
_sc_params = pltpu.CompilerParams(use_tc_tiling_on_sc=False)


def _deg_body(dstb, zdeg, ones_hbm, out, idx_d, ones_v, acc):
    c = lax.axis_index("c")
    s = lax.axis_index("s")
    w = c * NS + s
    r0 = s * RPT
    pltpu.sync_copy(zdeg.at[pl.ds(r0, RPT)], acc.at[pl.ds(r0, RPT)])
    pltpu.sync_copy(ones_hbm, ones_v)
    pltpu.sync_copy(dstb.at[pl.ds(w * BPW, BPW)], idx_d)
    plsc.subcore_barrier()

    def body(j, carry):
        pltpu.sync_copy(ones_v, acc.at[idx_d.at[j]], add=True)
        return carry

    lax.fori_loop(0, BPW, body, 0, unroll=False)
    plsc.subcore_barrier()
    pltpu.sync_copy(acc.at[pl.ds(r0, RPT)], out.at[c, pl.ds(r0, RPT)])


_deg_call = functools.partial(
    pl.kernel,
    out_type=jax.ShapeDtypeStruct((NC, NROWS, DEGW), _bf16),
    mesh=_mesh,
    compiler_params=_sc_params,
    scratch_types=[
        pltpu.VMEM((BPW, BLK), jnp.int32),
        pltpu.VMEM((BLK, DEGW), _bf16),
        pltpu.VMEM_SHARED((NROWS, DEGW), _bf16),
    ],
)(_deg_body)


BPW0 = 7
BPW1 = BPW * 2 - BPW0
BPWMAX = max(BPW0, BPW1)


def _agg_body(g_hbm, srcb, dstb, zacc, out, idx_s, idx_d, rows, acc, sem):
    c = lax.axis_index("c")
    s = lax.axis_index("s")
    r0 = s * RPT
    b0 = jnp.where(c == 0, s * BPW0, NS * BPW0 + s * BPW1)
    nb = jnp.where(c == 0, BPW0, BPW1)
    pltpu.sync_copy(zacc.at[pl.ds(r0, RPT)], acc.at[pl.ds(r0, RPT)])
    pltpu.sync_copy(srcb.at[pl.ds(b0, BPWMAX)], idx_s)
    pltpu.sync_copy(dstb.at[pl.ds(b0, BPWMAX)], idx_d)
    plsc.subcore_barrier()

    def body(j, carry):
        pltpu.async_copy(g_hbm.at[idx_s.at[j]], rows, sem).wait()
        pltpu.sync_copy(rows, acc.at[idx_d.at[j]], add=True)
        return carry

    lax.fori_loop(0, nb, body, 0, unroll=False)
    plsc.subcore_barrier()
    pltpu.sync_copy(acc.at[pl.ds(r0, RPT)], out.at[c, pl.ds(r0, RPT)])


_agg_call = functools.partial(
    pl.kernel,
    out_type=jax.ShapeDtypeStruct((NC, NROWS, D_H), _bf16),
    mesh=_mesh,
    compiler_params=_sc_params,
    scratch_types=[
        pltpu.VMEM((BPWMAX, BLK), jnp.int32),
        pltpu.VMEM((BPWMAX, BLK), jnp.int32),
        pltpu.VMEM((BLK, D_H), _bf16),
        pltpu.VMEM_SHARED((NROWS, D_H), _bf16),
        pltpu.SemaphoreType.DMA,
    ],
)(_agg_body)


def _g_body(x_ref, w_ref, degp_ref, g_ref, dinv_ref):
    dsum = degp_ref[0].astype(_f32) + degp_ref[1].astype(_f32)
    deg = jnp.sum(dsum[:N], axis=1, keepdims=True) * (1.0 / DEGW) + 1.0
    dinv = lax.rsqrt(jnp.maximum(deg, 1.0))
    dinv_ref[...] = dinv
    h1 = jnp.dot(x_ref[...], w_ref[...], preferred_element_type=_f32)
    g_ref[...] = (h1 * dinv).astype(_bf16)


def _g_call(x, W1, degp):
    return pl.pallas_call(
        _g_body,
        out_shape=(
            jax.ShapeDtypeStruct((N, D_H), _bf16),
            jax.ShapeDtypeStruct((N, 1), _f32),
        ),
    )(x, W1, degp)


def _mid_body(p_ref, g_ref, dinv_ref, b_ref, w_ref, o_ref):
    p = p_ref[0].astype(_f32)[:N] + p_ref[1].astype(_f32)[:N]
    agg = p + g_ref[...].astype(_f32)
    out1 = agg * dinv_ref[...] + b_ref[...]
    h2 = jnp.maximum(out1, 0.0)
    g2 = jnp.dot(h2, w_ref[...], preferred_element_type=_f32) * dinv_ref[...]
    o_ref[...] = g2.astype(_bf16)


def _mid_call(p1, g1, dinv, b1, W2):
    return pl.pallas_call(
        _mid_body,
        out_shape=jax.ShapeDtypeStruct((N, D_H), _bf16),
    )(p1, g1, dinv, b1.reshape(1, D_H), W2)


def _out_body(p_ref, g_ref, dinv_ref, b_ref, o_ref):
    p = p_ref[0].astype(_f32)[:N] + p_ref[1].astype(_f32)[:N]
    agg = p + g_ref[...].astype(_f32)
    o_ref[...] = agg * dinv_ref[...] + b_ref[...]


def _out_call(p2, g2, dinv, b2):
    return pl.pallas_call(
        _out_body,
        out_shape=jax.ShapeDtypeStruct((N, D_H), _f32),
    )(p2, g2, dinv, b2.reshape(1, D_H))


def kernel(x, edge_index, W1, b1, W2, b2):
    pad = EPAD - E
    srcp = jnp.concatenate(
        [edge_index[0], jnp.zeros((pad,), jnp.int32)]).reshape(TOTB, BLK)
    trash = TRASH + jnp.arange(pad, dtype=jnp.int32) % (NROWS - N)
    dstp = jnp.concatenate([edge_index[1], trash]).reshape(TOTB, BLK)
    zacc = jnp.zeros((NROWS, D_H), _bf16)
    zdeg = jnp.zeros((NROWS, DEGW), _bf16)
    ones = jnp.ones((BLK, DEGW), _bf16)

    degp = _deg_call(dstp, zdeg, ones)
    g1, dinv = _g_call(x, W1, degp)
    p1 = _agg_call(g1, srcp, dstp, zacc)
    g2 = _mid_call(p1, g1, dinv, b1, W2)
    p2 = _agg_call(g2, srcp, dstp, zacc)
    return _out_call(p2, g2, dinv, b2)
